# Initial kernel scaffold; baseline (speedup 1.0000x reference)
import jax
import jax.numpy as jnp
from jax.experimental import pallas as pl


def kernel(x, W1, b1, W2, b2, Wg, bg):
    raise NotImplementedError("write your pallas kernel here")


# speedup vs baseline: 1.3601x; 1.3601x over previous
import jax
import jax.numpy as jnp
from jax.experimental import pallas as pl
from jax.experimental.pallas import tpu as pltpu

D_MODEL = 1024
D_FF = 2048
N_EXPERTS = 8
N_TOKENS = 2048


def _gating_kernel(logits_ref, w_ref):
    logits = logits_ref[...]
    e = jax.lax.broadcasted_iota(jnp.int32, logits.shape, 1)
    l1 = jnp.max(logits, axis=1, keepdims=True)
    i1 = jnp.min(jnp.where(logits == l1, e, N_EXPERTS), axis=1, keepdims=True)
    m1 = e == i1
    masked = jnp.where(m1, -jnp.inf, logits)
    l2 = jnp.max(masked, axis=1, keepdims=True)
    i2 = jnp.min(jnp.where(masked == l2, e, N_EXPERTS), axis=1, keepdims=True)
    m2 = e == i2
    t = jnp.exp(l2 - l1)
    w2 = t / (1.0 + t)
    w1 = 1.0 - w2
    w_ref[...] = jnp.where(m1, w1, 0.0) + jnp.where(m2, w2, 0.0)


def _ffn_kernel(x_ref, w_ref, w1_ref, b1_ref, w2_ref, b2_ref, out_ref):
    e = pl.program_id(1)
    xb = x_ref[...].astype(jnp.bfloat16)
    h = jnp.dot(xb, w1_ref[0].astype(jnp.bfloat16),
                preferred_element_type=jnp.float32)
    h = jnp.maximum(h + b1_ref[0], 0.0).astype(jnp.bfloat16)
    o = jnp.dot(h, w2_ref[0].astype(jnp.bfloat16),
                preferred_element_type=jnp.float32)
    o = o + b2_ref[0]
    wv = w_ref[...]
    eids = jax.lax.broadcasted_iota(jnp.int32, wv.shape, 1)
    gate = jnp.sum(jnp.where(eids == e, wv, 0.0), axis=1, keepdims=True)
    contrib = o * gate

    @pl.when(e == 0)
    def _():
        out_ref[...] = contrib

    @pl.when(e != 0)
    def _():
        out_ref[...] += contrib


def kernel(x, W1, b1, W2, b2, Wg, bg):
    n = x.shape[0]
    logits = x @ Wg + bg
    w = pl.pallas_call(
        _gating_kernel,
        out_shape=jax.ShapeDtypeStruct((n, N_EXPERTS), jnp.float32),
        in_specs=[pl.BlockSpec((n, N_EXPERTS), lambda: (0, 0))],
        out_specs=pl.BlockSpec((n, N_EXPERTS), lambda: (0, 0)),
    )(logits)

    bm = 512
    grid = (n // bm, N_EXPERTS)
    out = pl.pallas_call(
        _ffn_kernel,
        grid=grid,
        out_shape=jax.ShapeDtypeStruct((n, D_MODEL), jnp.float32),
        in_specs=[
            pl.BlockSpec((bm, D_MODEL), lambda t, e: (t, 0)),
            pl.BlockSpec((bm, N_EXPERTS), lambda t, e: (t, 0)),
            pl.BlockSpec((1, D_MODEL, D_FF), lambda t, e: (e, 0, 0)),
            pl.BlockSpec((1, 1, D_FF), lambda t, e: (e, 0, 0)),
            pl.BlockSpec((1, D_FF, D_MODEL), lambda t, e: (e, 0, 0)),
            pl.BlockSpec((1, 1, D_MODEL), lambda t, e: (e, 0, 0)),
        ],
        out_specs=pl.BlockSpec((bm, D_MODEL), lambda t, e: (t, 0)),
    )(x, w, W1, b1.reshape(N_EXPERTS, 1, D_FF), W2,
      b2.reshape(N_EXPERTS, 1, D_MODEL))
    return out
